# speedup vs baseline: 2.9739x; 2.9739x over previous
import jax
import jax.numpy as jnp
from jax import lax
from jax.experimental import pallas as pl
from jax.experimental.pallas import tpu as pltpu
from jax.experimental.pallas import tpu_sc as plsc

NC, NS = 2, 16
NW = NC * NS
CHUNK = 128


def _gather_body(x_hbm, idx_hbm, out_hbm, idx_v, rows_v, sem):
    wid = lax.axis_index("s") * NC + lax.axis_index("c")
    n_chunks = idx_v.shape[0]
    base = wid * (n_chunks * CHUNK)
    pltpu.sync_copy(idx_hbm.at[wid], idx_v)

    def body(g, carry):
        pltpu.async_copy(x_hbm.at[idx_v.at[g]], rows_v, sem).wait()
        pltpu.sync_copy(rows_v, out_hbm.at[pl.ds(base + g * CHUNK, CHUNK)])
        return carry

    lax.fori_loop(0, n_chunks, body, 0)


def kernel(x, index):
    B, K = index.shape
    D = x.shape[1]
    total = B * K
    n_chunks = total // (NW * CHUNK)
    idx = index.reshape(-1).astype(jnp.int32).reshape(NW, n_chunks, CHUNK)

    gather = pl.kernel(
        _gather_body,
        out_type=jax.ShapeDtypeStruct((total, D), x.dtype),
        mesh=plsc.VectorSubcoreMesh(core_axis_name="c", subcore_axis_name="s"),
        scratch_types=[
            pltpu.VMEM((n_chunks, CHUNK), jnp.int32),
            pltpu.VMEM((CHUNK, D), jnp.float32),
            pltpu.SemaphoreType.DMA,
        ],
    )
    out_flat = gather(x, idx)
    return out_flat.reshape(B, K, D)


# speedup vs baseline: 3.3421x; 1.1238x over previous
import jax
import jax.numpy as jnp
from jax import lax
from jax.experimental import pallas as pl
from jax.experimental.pallas import tpu as pltpu
from jax.experimental.pallas import tpu_sc as plsc

NC, NS = 2, 16
NW = NC * NS
CHUNK = 128


NBUF = 5


def _gather_body(x_hbm, idx_hbm, out_hbm, idx_v, rows_v, gsems, ssems):
    wid = lax.axis_index("s") * NC + lax.axis_index("c")
    n_chunks = idx_v.shape[0]
    base = wid * (n_chunks * CHUNK)
    pltpu.sync_copy(idx_hbm.at[wid], idx_v)

    def gather_copy(g, b):
        return pltpu.make_async_copy(
            x_hbm.at[idx_v.at[g]], rows_v.at[b], gsems.at[b])

    def store_copy(g, b):
        return pltpu.make_async_copy(
            rows_v.at[b], out_hbm.at[pl.ds(base + g * CHUNK, CHUNK)],
            ssems.at[b])

    for b in range(NBUF):
        gather_copy(b, b).start()

    n_groups = n_chunks // NBUF

    def group(o, carry):
        for b in range(NBUF):
            g = o * NBUF + b
            gather_copy(g, b).wait()
            store_copy(g, b).start()
            store_copy(g, b).wait()
            gather_copy(g + NBUF, b).start()
        return carry

    lax.fori_loop(0, n_groups - 1, group, 0)

    for b in range(NBUF):
        g = (n_groups - 1) * NBUF + b
        gather_copy(g, b).wait()
        store_copy(g, b).start()
        store_copy(g, b).wait()


def kernel(x, index):
    B, K = index.shape
    D = x.shape[1]
    total = B * K
    n_chunks = total // (NW * CHUNK)
    idx = index.reshape(-1).astype(jnp.int32).reshape(NW, n_chunks, CHUNK)

    gather = pl.kernel(
        _gather_body,
        out_type=jax.ShapeDtypeStruct((total, D), x.dtype),
        mesh=plsc.VectorSubcoreMesh(core_axis_name="c", subcore_axis_name="s"),
        scratch_types=[
            pltpu.VMEM((n_chunks, CHUNK), jnp.int32),
            pltpu.VMEM((NBUF, CHUNK, D), jnp.float32),
            pltpu.SemaphoreType.DMA((NBUF,)),
            pltpu.SemaphoreType.DMA((NBUF,)),
        ],
    )
    out_flat = gather(x, idx)
    return out_flat.reshape(B, K, D)


# speedup vs baseline: 5.9770x; 1.7884x over previous
import jax
import jax.numpy as jnp
from jax import lax
from jax.experimental import pallas as pl
from jax.experimental.pallas import tpu as pltpu
from jax.experimental.pallas import tpu_sc as plsc

NC, NS = 2, 16
NW = NC * NS
NBUF = 8


def _gather_body(x_hbm, idx_hbm, out_hbm, idx_v, rows_v, gsems, ssems):
    wid = lax.axis_index("s") * NC + lax.axis_index("c")
    n_chunks, K = idx_v.shape
    base = wid * n_chunks
    pltpu.sync_copy(idx_hbm.at[wid], idx_v)

    def gather_copy(g, b):
        return pltpu.make_async_copy(
            x_hbm.at[idx_v.at[g]], rows_v.at[b], gsems.at[b])

    def store_copy(g, b):
        return pltpu.make_async_copy(
            rows_v.at[b], out_hbm.at[base + g], ssems.at[b])

    for b in range(NBUF):
        gather_copy(b, b).start()

    n_groups = n_chunks // NBUF

    def group(o, carry):
        for b in range(NBUF):
            g = o * NBUF + b
            gather_copy(g, b).wait()
            store_copy(g, b).start()
            store_copy(g, b).wait()
            gather_copy(g + NBUF, b).start()
        return carry

    lax.fori_loop(0, n_groups - 1, group, 0)

    for b in range(NBUF):
        g = (n_groups - 1) * NBUF + b
        gather_copy(g, b).wait()
        store_copy(g, b).start()
        store_copy(g, b).wait()


def kernel(x, index):
    B, K = index.shape
    D = x.shape[1]
    n_chunks = B // NW
    idx = index.astype(jnp.int32).reshape(NW, n_chunks, K)

    gather = pl.kernel(
        _gather_body,
        out_type=jax.ShapeDtypeStruct((B, K, D), x.dtype),
        mesh=plsc.VectorSubcoreMesh(core_axis_name="c", subcore_axis_name="s"),
        scratch_types=[
            pltpu.VMEM((n_chunks, K), jnp.int32),
            pltpu.VMEM((NBUF, K, D), jnp.float32),
            pltpu.SemaphoreType.DMA((NBUF,)),
            pltpu.SemaphoreType.DMA((NBUF,)),
        ],
    )
    return gather(x, idx)


# speedup vs baseline: 5.9774x; 1.0001x over previous
import jax
import jax.numpy as jnp
from jax import lax
from jax.experimental import pallas as pl
from jax.experimental.pallas import tpu as pltpu
from jax.experimental.pallas import tpu_sc as plsc

NC, NS = 2, 16
NW = NC * NS
NBUF = 8


def _gather_body(x_hbm, idx_hbm, out_hbm, idx_v, rows_v, gsems, ssems):
    wid = lax.axis_index("s") * NC + lax.axis_index("c")
    n_chunks, K = idx_v.shape
    base = wid * n_chunks
    pltpu.sync_copy(idx_hbm.at[wid], idx_v)

    def gather_copy(g, b):
        return pltpu.make_async_copy(
            x_hbm.at[idx_v.at[g]], rows_v.at[b], gsems.at[b])

    def store_copy(g, b):
        return pltpu.make_async_copy(
            rows_v.at[b], out_hbm.at[base + g], ssems.at[b])

    for b in range(NBUF):
        gather_copy(b, b).start()

    n_groups = n_chunks // NBUF

    def group(o, carry):
        for b in range(NBUF):
            g = o * NBUF + b
            gather_copy(g, b).wait()
            store_copy(g, b).start()
            store_copy(g, b).wait()
            gather_copy(g + NBUF, b).start()
        return carry

    lax.fori_loop(0, n_groups - 1, group, 0)

    for b in range(NBUF):
        g = (n_groups - 1) * NBUF + b
        gather_copy(g, b).wait()
        store_copy(g, b).start()
        store_copy(g, b).wait()


def kernel(x, index):
    B, K = index.shape
    D = x.shape[1]
    n_chunks = B // NW
    idx = index.astype(jnp.int32).reshape(NW, n_chunks, K)

    gather = pl.kernel(
        _gather_body,
        out_type=jax.ShapeDtypeStruct((B, K, D), x.dtype),
        mesh=plsc.VectorSubcoreMesh(core_axis_name="c", subcore_axis_name="s"),
        compiler_params=pltpu.CompilerParams(use_tc_tiling_on_sc=True),
        scratch_types=[
            pltpu.VMEM((n_chunks, K), jnp.int32),
            pltpu.VMEM((NBUF, K, D), jnp.float32),
            pltpu.SemaphoreType.DMA((NBUF,)),
            pltpu.SemaphoreType.DMA((NBUF,)),
        ],
    )
    return gather(x, idx)


# speedup vs baseline: 10.3959x; 1.7392x over previous
import jax
import jax.numpy as jnp
from jax import lax
from jax.experimental import pallas as pl
from jax.experimental.pallas import tpu as pltpu
from jax.experimental.pallas import tpu_sc as plsc

NC, NS = 2, 16
NW = NC * NS
CHUNK = 128
NBUF = 5


def _gather_body(x_hbm, idx_hbm, out_hbm, idx_v, rows_v, gsems, ssems):
    wid = lax.axis_index("s") * NC + lax.axis_index("c")
    n_chunks = idx_v.shape[0]
    base = wid * (n_chunks * CHUNK)
    pltpu.sync_copy(idx_hbm.at[wid], idx_v)

    def gather_copy(g, b):
        return pltpu.make_async_copy(
            x_hbm.at[idx_v.at[g]], rows_v.at[b], gsems.at[b])

    def store_copy(g, b):
        return pltpu.make_async_copy(
            rows_v.at[b], out_hbm.at[pl.ds(base + g * CHUNK, CHUNK)],
            ssems.at[b])

    for b in range(NBUF):
        gather_copy(b, b).start()

    n_groups = n_chunks // NBUF

    def group(o, carry):
        for b in range(NBUF):
            g = o * NBUF + b
            gather_copy(g, b).wait()
            store_copy(g, b).start()
            store_copy(g, b).wait()
            gather_copy(g + NBUF, b).start()
        return carry

    lax.fori_loop(0, n_groups - 1, group, 0)

    for b in range(NBUF):
        g = (n_groups - 1) * NBUF + b
        gather_copy(g, b).wait()
        store_copy(g, b).start()
        store_copy(g, b).wait()


def kernel(x, index):
    B, K = index.shape
    D = x.shape[1]
    total = B * K
    n_chunks = total // (NW * CHUNK)
    idx = jnp.swapaxes(index, 0, 1).astype(jnp.int32).reshape(
        NW, n_chunks, CHUNK)

    gather = pl.kernel(
        _gather_body,
        out_type=jax.ShapeDtypeStruct((total, D), x.dtype),
        mesh=plsc.VectorSubcoreMesh(core_axis_name="c", subcore_axis_name="s",
                                    num_cores=NC, num_subcores=NS),
        scratch_types=[
            pltpu.VMEM((n_chunks, CHUNK), jnp.int32),
            pltpu.VMEM((NBUF, CHUNK, D), jnp.float32),
            pltpu.SemaphoreType.DMA((NBUF,)),
            pltpu.SemaphoreType.DMA((NBUF,)),
        ],
    )
    out_flat = gather(x, idx)
    return jnp.swapaxes(out_flat.reshape(K, B, D), 0, 1)


# speedup vs baseline: 10.6650x; 1.0259x over previous
import jax
import jax.numpy as jnp
from jax import lax
from jax.experimental import pallas as pl
from jax.experimental.pallas import tpu as pltpu
from jax.experimental.pallas import tpu_sc as plsc

NC, NS = 2, 16
NW = NC * NS
CHUNK = 128
NBUF = 5


def _gather_body(x_hbm, idx_hbm, out_hbm, idx_v, rows_v, gsems, ssems):
    wid = lax.axis_index("s") * NC + lax.axis_index("c")
    K, B = idx_hbm.shape
    col0 = wid * CHUNK
    pltpu.sync_copy(idx_hbm.at[:, pl.ds(col0, CHUNK)], idx_v)

    def gather_copy(g, b):
        return pltpu.make_async_copy(
            x_hbm.at[idx_v.at[g]], rows_v.at[b], gsems.at[b])

    def store_copy(g, b):
        return pltpu.make_async_copy(
            rows_v.at[b], out_hbm.at[pl.ds(g * B + col0, CHUNK)],
            ssems.at[b])

    for b in range(NBUF):
        gather_copy(b, b).start()

    n_groups = K // NBUF

    def group(o, carry):
        for b in range(NBUF):
            g = o * NBUF + b
            gather_copy(g, b).wait()
            store_copy(g, b).start()
            store_copy(g, b).wait()
            gather_copy(g + NBUF, b).start()
        return carry

    lax.fori_loop(0, n_groups - 1, group, 0)

    for b in range(NBUF):
        g = (n_groups - 1) * NBUF + b
        gather_copy(g, b).wait()
        store_copy(g, b).start()
        store_copy(g, b).wait()


def kernel(x, index):
    B, K = index.shape
    D = x.shape[1]
    idx_t = jnp.swapaxes(index, 0, 1).astype(jnp.int32)

    gather = pl.kernel(
        _gather_body,
        out_type=jax.ShapeDtypeStruct((K * B, D), x.dtype),
        mesh=plsc.VectorSubcoreMesh(core_axis_name="c", subcore_axis_name="s",
                                    num_cores=NC, num_subcores=NS),
        scratch_types=[
            pltpu.VMEM((K, CHUNK), jnp.int32),
            pltpu.VMEM((NBUF, CHUNK, D), jnp.float32),
            pltpu.SemaphoreType.DMA((NBUF,)),
            pltpu.SemaphoreType.DMA((NBUF,)),
        ],
    )
    out_flat = gather(x, idx_t)
    return jnp.swapaxes(out_flat.reshape(K, B, D), 0, 1)
